# baseline (speedup 1.0000x reference)
import functools

import jax
import jax.numpy as jnp
from jax import lax
from jax.experimental import pallas as pl
from jax.experimental.pallas import tpu as pltpu
from jax.experimental.pallas import tpu_sc as plsc

NC = 2
NS = 16
NW = NC * NS
BLK = 16
MBLK = 32
DEG_W = 16


def _mesh():
    return plsc.VectorSubcoreMesh(core_axis_name="c", subcore_axis_name="s")


def _deg_call(dst2, zeros_s, ones_w, n_pad, stripe):
    nwin = dst2.shape[1]
    win = dst2.shape[2]
    nblk = nwin // BLK

    @functools.partial(
        pl.kernel,
        out_type=jax.ShapeDtypeStruct((NC, n_pad, DEG_W), jnp.float32),
        mesh=_mesh(),
        compiler_params=pltpu.CompilerParams(use_tc_tiling_on_sc=False),
        scratch_types=[
            pltpu.VMEM((BLK, win), jnp.int32),
            pltpu.VMEM((win, DEG_W), jnp.float32),
            pltpu.VMEM_SHARED((n_pad, DEG_W), jnp.float32),
            pltpu.SemaphoreType.DMA,
        ],
    )
    def deg_kernel(dst_hbm, zeros_hbm, ones_hbm, out_hbm,
                   didx_blk, ones_v, acc, ssem):
        cid = lax.axis_index("c")
        sid = lax.axis_index("s")
        wid = sid * NC + cid
        pltpu.sync_copy(ones_hbm, ones_v)
        pltpu.sync_copy(zeros_hbm, acc.at[pl.ds(sid * stripe, stripe)])
        plsc.subcore_barrier()

        @pl.loop(0, nblk)
        def _(b):
            pltpu.sync_copy(dst_hbm.at[wid, pl.ds(b * BLK, BLK)], didx_blk)

            @pl.loop(0, BLK)
            def _(j):
                pltpu.async_copy(ones_v, acc.at[didx_blk.at[j]], ssem,
                                 add=True)

            @pl.loop(0, BLK)
            def _(j):
                pltpu.make_async_copy(ones_v, acc.at[didx_blk.at[j]],
                                      ssem).wait()

        plsc.subcore_barrier()
        pltpu.sync_copy(acc.at[pl.ds(sid * stripe, stripe)],
                        out_hbm.at[cid, pl.ds(sid * stripe, stripe)])

    return deg_kernel(dst2, zeros_s, ones_w)


def _msg_call(g, src2, dst2, zeros_s, n_pad, stripe, d, nbuf, nw0, nw1):
    win = src2.shape[1]

    @functools.partial(
        pl.kernel,
        out_type=jax.ShapeDtypeStruct((NC, n_pad, d), jnp.float32),
        mesh=_mesh(),
        compiler_params=pltpu.CompilerParams(use_tc_tiling_on_sc=False),
        scratch_types=(
            [pltpu.VMEM((MBLK, win), jnp.int32),
             pltpu.VMEM((MBLK, win), jnp.int32)]
            + [pltpu.VMEM((win, d), jnp.float32) for _ in range(nbuf)]
            + [pltpu.VMEM_SHARED((n_pad, d), jnp.float32)]
            + [pltpu.SemaphoreType.DMA for _ in range(2 * nbuf)]
        ),
    )
    def msg_kernel(g_hbm, src_hbm, dst_hbm, zeros_hbm, out_hbm,
                   sidx_blk, didx_blk, *scr):
        rows = scr[:nbuf]
        acc = scr[nbuf]
        gsem = scr[nbuf + 1:2 * nbuf + 1]
        ssem = scr[2 * nbuf + 1:]
        cid = lax.axis_index("c")
        sid = lax.axis_index("s")
        start = jnp.where(cid == 0, sid * nw0, 16 * nw0 + sid * nw1)
        nblk_me = jnp.where(cid == 0, nw0 // MBLK, nw1 // MBLK)
        pltpu.sync_copy(zeros_hbm, acc.at[pl.ds(sid * stripe, stripe)])
        plsc.subcore_barrier()

        @pl.loop(0, nblk_me)
        def _(b):
            w0 = start + b * MBLK
            pltpu.sync_copy(src_hbm.at[pl.ds(w0, MBLK)], sidx_blk)
            pltpu.sync_copy(dst_hbm.at[pl.ds(w0, MBLK)], didx_blk)
            for k in range(nbuf):
                pltpu.async_copy(g_hbm.at[sidx_blk.at[k]], rows[k], gsem[k])

            @pl.loop(0, MBLK, step=nbuf)
            def _(j):
                for k in range(nbuf):
                    pltpu.make_async_copy(g_hbm.at[sidx_blk.at[j + k]],
                                          rows[k], gsem[k]).wait()
                    pltpu.async_copy(rows[k], acc.at[didx_blk.at[j + k]],
                                     ssem[k], add=True)

                    @pl.when(j + k + nbuf < MBLK)
                    def _(k=k, j=j):
                        pltpu.make_async_copy(rows[k],
                                              acc.at[didx_blk.at[j + k]],
                                              ssem[k]).wait()
                        pltpu.async_copy(g_hbm.at[sidx_blk.at[j + k + nbuf]],
                                         rows[k], gsem[k])

            for k in range(nbuf):
                pltpu.make_async_copy(rows[k],
                                      acc.at[didx_blk.at[MBLK - nbuf + k]],
                                      ssem[k]).wait()

        plsc.subcore_barrier()
        pltpu.sync_copy(acc.at[pl.ds(sid * stripe, stripe)],
                        out_hbm.at[cid, pl.ds(sid * stripe, stripe)])

    return msg_kernel(g, src2, dst2, zeros_s)


def _mm(x, w):
    def body(x_ref, w_ref, o_ref):
        o_ref[...] = jnp.dot(x_ref[...], w_ref[...],
                             preferred_element_type=jnp.float32)

    return pl.pallas_call(
        body,
        out_shape=jax.ShapeDtypeStruct((x.shape[0], w.shape[1]), jnp.float32),
    )(x, w)


def _scale_call(degp, h):
    n_pad, d = h.shape

    def body(degp_ref, h_ref, g_ref, dis_ref):
        deg = degp_ref[0, :, 0:1] + degp_ref[1, :, 0:1] + 1.0
        dis = lax.rsqrt(deg)
        dis_ref[...] = dis
        g_ref[...] = dis * h_ref[...]

    return pl.pallas_call(
        body,
        out_shape=(jax.ShapeDtypeStruct((n_pad, d), jnp.float32),
                   jax.ShapeDtypeStruct((n_pad, 1), jnp.float32)),
    )(degp, h)


def _combine_mm_call(p, g, dis, b, w):
    n_pad, d = g.shape

    def body(p_ref, g_ref, dis_ref, b_ref, w_ref, o_ref):
        dis_v = dis_ref[...]
        s = dis_v * (p_ref[0] + p_ref[1] + g_ref[...]) + b_ref[...]
        act = jnp.maximum(s, 0.0)
        h = jnp.dot(act, w_ref[...], preferred_element_type=jnp.float32)
        o_ref[...] = dis_v * h

    return pl.pallas_call(
        body,
        out_shape=jax.ShapeDtypeStruct((n_pad, w.shape[1]), jnp.float32),
    )(p, g, dis, b, w)


def _head_call(p, g, dis, b, wfc, bfc):
    n_pad, d = g.shape

    def body(p_ref, g_ref, dis_ref, b_ref, w_ref, bfc_ref, o_ref):
        dis_v = dis_ref[...]
        s = dis_v * (p_ref[0] + p_ref[1] + g_ref[...]) + b_ref[...]
        act = jnp.maximum(s, 0.0)
        o_ref[...] = jnp.dot(act, w_ref[...],
                             preferred_element_type=jnp.float32) + bfc_ref[...]

    return pl.pallas_call(
        body,
        out_shape=jax.ShapeDtypeStruct((n_pad, wfc.shape[1]), jnp.float32),
    )(p, g, dis, b, wfc, bfc)


def _pad_edges_flat(edge_index, n, win, total_nwin):
    e = edge_index.shape[1]
    e_pad = total_nwin * win
    pad = e_pad - e
    src = jnp.concatenate(
        [edge_index[0].astype(jnp.int32), jnp.zeros((pad,), jnp.int32)])
    dst = jnp.concatenate(
        [edge_index[1].astype(jnp.int32), jnp.full((pad,), n, jnp.int32)])
    return src.reshape(total_nwin, win), dst.reshape(total_nwin, win)


def kernel(x, edge_index, W1, b1, W2, b2, Wfc, bfc):
    n, d_in = x.shape
    d_h1 = W1.shape[1]
    d_h2 = W2.shape[1]

    stripe_unit = NS * 8
    n_pad = ((n + 1 + stripe_unit - 1) // stripe_unit) * stripe_unit
    stripe = n_pad // NS

    e = edge_index.shape[1]
    nwt_a = -(-e // (16 * 64 * MBLK)) * MBLK
    nwt_b = -(-e // (16 * 128 * MBLK)) * MBLK
    src2a, dst2a = _pad_edges_flat(edge_index, n, 64, 16 * nwt_a)
    src2b, dst2b = _pad_edges_flat(edge_index, n, 128, 16 * nwt_b)
    nwin_deg = 16 * nwt_b // NW
    dst2deg = dst2b.reshape(NW, nwin_deg, 128)

    x_pad = jnp.pad(x, ((0, n_pad - n), (0, 0)))
    zeros16 = jnp.zeros((stripe, DEG_W), jnp.float32)
    ones16 = jnp.ones((128, DEG_W), jnp.float32)
    zeros_h1 = jnp.zeros((stripe, d_h1), jnp.float32)
    zeros_h2 = jnp.zeros((stripe, d_h2), jnp.float32)

    degp = _deg_call(dst2deg, zeros16, ones16, n_pad, stripe)
    h1 = _mm(x_pad, W1)
    g1, dis = _scale_call(degp, h1)

    p1 = _msg_call(g1, src2a, dst2a, zeros_h1, n_pad, stripe, d_h1, 4,
                   nwt_a, 0)
    g2 = _combine_mm_call(p1, g1, dis, b1.reshape(1, -1), W2)

    p2 = _msg_call(g2, src2b, dst2b, zeros_h2, n_pad, stripe, d_h2, 4,
                   0, nwt_b)
    y = _head_call(p2, g2, dis, b2.reshape(1, -1), Wfc, bfc.reshape(1, -1))

    return y[:n]


# speedup vs baseline: 2.5603x; 2.5603x over previous
import functools

import jax
import jax.numpy as jnp
from jax import lax
from jax.experimental import pallas as pl
from jax.experimental.pallas import tpu as pltpu
from jax.experimental.pallas import tpu_sc as plsc

NC = 2
NS = 16
NW = NC * NS
BLK = 16
MBLK = 32
DEG_W = 16


def _mesh():
    return plsc.VectorSubcoreMesh(core_axis_name="c", subcore_axis_name="s")


def _deg_call(dst2, zeros_s, ones_w, n_pad, stripe):
    nwin = dst2.shape[1]
    win = dst2.shape[2]
    nblk = nwin // BLK

    @functools.partial(
        pl.kernel,
        out_type=jax.ShapeDtypeStruct((NC, n_pad, DEG_W), jnp.float32),
        mesh=_mesh(),
        compiler_params=pltpu.CompilerParams(use_tc_tiling_on_sc=False),
        scratch_types=[
            pltpu.VMEM((BLK, win), jnp.int32),
            pltpu.VMEM((win, DEG_W), jnp.float32),
            pltpu.VMEM_SHARED((n_pad, DEG_W), jnp.float32),
            pltpu.SemaphoreType.DMA,
        ],
    )
    def deg_kernel(dst_hbm, zeros_hbm, ones_hbm, out_hbm,
                   didx_blk, ones_v, acc, ssem):
        cid = lax.axis_index("c")
        sid = lax.axis_index("s")
        wid = sid * NC + cid
        pltpu.sync_copy(ones_hbm, ones_v)
        pltpu.sync_copy(zeros_hbm, acc.at[pl.ds(sid * stripe, stripe)])
        plsc.subcore_barrier()

        @pl.loop(0, nblk)
        def _(b):
            pltpu.sync_copy(dst_hbm.at[wid, pl.ds(b * BLK, BLK)], didx_blk)

            @pl.loop(0, BLK)
            def _(j):
                pltpu.async_copy(ones_v, acc.at[didx_blk.at[j]], ssem,
                                 add=True)

            @pl.loop(0, BLK)
            def _(j):
                pltpu.make_async_copy(ones_v, acc.at[didx_blk.at[j]],
                                      ssem).wait()

        plsc.subcore_barrier()
        pltpu.sync_copy(acc.at[pl.ds(sid * stripe, stripe)],
                        out_hbm.at[cid, pl.ds(sid * stripe, stripe)])

    return deg_kernel(dst2, zeros_s, ones_w)


def _msg_call(g, src2, dst2, zeros_s, n_pad, stripe, d, nbuf):
    win = src2.shape[1]
    nwt = src2.shape[0] // NS
    dh = d // NC

    @functools.partial(
        pl.kernel,
        out_type=jax.ShapeDtypeStruct((n_pad, d), jnp.float32),
        mesh=_mesh(),
        compiler_params=pltpu.CompilerParams(use_tc_tiling_on_sc=False),
        scratch_types=(
            [pltpu.VMEM((MBLK, win), jnp.int32),
             pltpu.VMEM((MBLK, win), jnp.int32)]
            + [pltpu.VMEM((win, dh), jnp.float32) for _ in range(nbuf)]
            + [pltpu.VMEM_SHARED((n_pad, dh), jnp.float32),
               pltpu.VMEM_SHARED((n_pad, dh), jnp.float32)]
            + [pltpu.SemaphoreType.DMA for _ in range(2 * nbuf)]
        ),
    )
    def msg_kernel(g_hbm, src_hbm, dst_hbm, zeros_hbm, out_hbm,
                   sidx_blk, didx_blk, *scr):
        rows = scr[:nbuf]
        acc = scr[nbuf]
        staged = scr[nbuf + 1]
        gsem = scr[nbuf + 2:nbuf + 2 + nbuf]
        ssem = scr[nbuf + 2 + nbuf:]
        cid = lax.axis_index("c")
        sid = lax.axis_index("s")
        rs = pl.ds(sid * stripe, stripe)
        cs = pl.ds(cid * dh, dh)
        pltpu.sync_copy(zeros_hbm, acc.at[rs])
        pltpu.sync_copy(g_hbm.at[rs, cs], staged.at[rs])
        plsc.subcore_barrier()

        @pl.loop(0, nwt // MBLK)
        def _(b):
            w0 = sid * nwt + b * MBLK
            pltpu.sync_copy(src_hbm.at[pl.ds(w0, MBLK)], sidx_blk)
            pltpu.sync_copy(dst_hbm.at[pl.ds(w0, MBLK)], didx_blk)
            for k in range(nbuf):
                pltpu.async_copy(staged.at[sidx_blk.at[k]], rows[k], gsem[k])

            @pl.loop(0, MBLK, step=nbuf)
            def _(j):
                for k in range(nbuf):
                    pltpu.make_async_copy(staged.at[sidx_blk.at[j + k]],
                                          rows[k], gsem[k]).wait()
                    pltpu.async_copy(rows[k], acc.at[didx_blk.at[j + k]],
                                     ssem[k], add=True)

                    @pl.when(j + k + nbuf < MBLK)
                    def _(k=k, j=j):
                        pltpu.make_async_copy(rows[k],
                                              acc.at[didx_blk.at[j + k]],
                                              ssem[k]).wait()
                        pltpu.async_copy(staged.at[sidx_blk.at[j + k + nbuf]],
                                         rows[k], gsem[k])

            for k in range(nbuf):
                pltpu.make_async_copy(rows[k],
                                      acc.at[didx_blk.at[MBLK - nbuf + k]],
                                      ssem[k]).wait()

        plsc.subcore_barrier()
        pltpu.sync_copy(acc.at[rs], out_hbm.at[rs, cs])

    return msg_kernel(g, src2, dst2, zeros_s)


def _mm(x, w):
    def body(x_ref, w_ref, o_ref):
        o_ref[...] = jnp.dot(x_ref[...], w_ref[...],
                             preferred_element_type=jnp.float32)

    return pl.pallas_call(
        body,
        out_shape=jax.ShapeDtypeStruct((x.shape[0], w.shape[1]), jnp.float32),
    )(x, w)


def _scale_call(degp, h):
    n_pad, d = h.shape

    def body(degp_ref, h_ref, g_ref, dis_ref):
        deg = degp_ref[0, :, 0:1] + degp_ref[1, :, 0:1] + 1.0
        dis = lax.rsqrt(deg)
        dis_ref[...] = dis
        g_ref[...] = dis * h_ref[...]

    return pl.pallas_call(
        body,
        out_shape=(jax.ShapeDtypeStruct((n_pad, d), jnp.float32),
                   jax.ShapeDtypeStruct((n_pad, 1), jnp.float32)),
    )(degp, h)


def _combine_mm_call(p, g, dis, b, w):
    n_pad, d = g.shape

    def body(p_ref, g_ref, dis_ref, b_ref, w_ref, o_ref):
        dis_v = dis_ref[...]
        s = dis_v * (p_ref[...] + g_ref[...]) + b_ref[...]
        act = jnp.maximum(s, 0.0)
        h = jnp.dot(act, w_ref[...], preferred_element_type=jnp.float32)
        o_ref[...] = dis_v * h

    return pl.pallas_call(
        body,
        out_shape=jax.ShapeDtypeStruct((n_pad, w.shape[1]), jnp.float32),
    )(p, g, dis, b, w)


def _head_call(p, g, dis, b, wfc, bfc):
    n_pad, d = g.shape

    def body(p_ref, g_ref, dis_ref, b_ref, w_ref, bfc_ref, o_ref):
        dis_v = dis_ref[...]
        s = dis_v * (p_ref[...] + g_ref[...]) + b_ref[...]
        act = jnp.maximum(s, 0.0)
        o_ref[...] = jnp.dot(act, w_ref[...],
                             preferred_element_type=jnp.float32) + bfc_ref[...]

    return pl.pallas_call(
        body,
        out_shape=jax.ShapeDtypeStruct((n_pad, wfc.shape[1]), jnp.float32),
    )(p, g, dis, b, wfc, bfc)


def _pad_edges_flat(edge_index, n, win, total_nwin):
    e = edge_index.shape[1]
    e_pad = total_nwin * win
    pad = e_pad - e
    src = jnp.concatenate(
        [edge_index[0].astype(jnp.int32), jnp.zeros((pad,), jnp.int32)])
    dst = jnp.concatenate(
        [edge_index[1].astype(jnp.int32), jnp.full((pad,), n, jnp.int32)])
    return src.reshape(total_nwin, win), dst.reshape(total_nwin, win)


def kernel(x, edge_index, W1, b1, W2, b2, Wfc, bfc):
    n, d_in = x.shape
    d_h1 = W1.shape[1]
    d_h2 = W2.shape[1]

    stripe_unit = NS * 8
    n_pad = ((n + 1 + stripe_unit - 1) // stripe_unit) * stripe_unit
    stripe = n_pad // NS

    e = edge_index.shape[1]
    nwt = -(-e // (NS * 128 * MBLK)) * MBLK
    src2, dst2 = _pad_edges_flat(edge_index, n, 128, NS * nwt)
    dst2deg = dst2.reshape(NW, NS * nwt // NW, 128)

    x_pad = jnp.pad(x, ((0, n_pad - n), (0, 0)))
    zeros16 = jnp.zeros((stripe, DEG_W), jnp.float32)
    ones16 = jnp.ones((128, DEG_W), jnp.float32)
    zeros_h1 = jnp.zeros((stripe, d_h1 // NC), jnp.float32)
    zeros_h2 = jnp.zeros((stripe, d_h2 // NC), jnp.float32)

    degp = _deg_call(dst2deg, zeros16, ones16, n_pad, stripe)
    h1 = _mm(x_pad, W1)
    g1, dis = _scale_call(degp, h1)

    p1 = _msg_call(g1, src2, dst2, zeros_h1, n_pad, stripe, d_h1, 4)
    g2 = _combine_mm_call(p1, g1, dis, b1.reshape(1, -1), W2)

    p2 = _msg_call(g2, src2, dst2, zeros_h2, n_pad, stripe, d_h2, 4)
    y = _head_call(p2, g2, dis, b2.reshape(1, -1), Wfc, bfc.reshape(1, -1))

    return y[:n]
